# baseline (speedup 1.0000x reference)
import functools

import jax
import jax.numpy as jnp
from jax import lax
from jax.experimental import pallas as pl
from jax.experimental.pallas import tpu as pltpu
from jax.experimental.pallas import tpu_sc as plsc

_PAD = 1
_B = 4
_S = 8192
_D = 1024
_TOT = _B * _S
_NROW = 16384

_NC = 2
_NS = 16
_NW = _NC * _NS
_PER_W = _TOT // _NW
_WPR = _S // _PER_W
_L = 16
_SEG = _PER_W // _L
_PRE = (_WPR - 1) * _PER_W
_CH = 16
_NBUF = 3
_CHUNKS = [(i * _CH, _CH) for i in range(_PER_W // _CH)]
if _PER_W % _CH:
    _CHUNKS.append((_PER_W - _PER_W % _CH, _PER_W % _CH))


def _lane_prefix(x, tmp_v, lane, zeros):
    cs = x
    for k in (1, 2, 4, 8):
        tmp_v[...] = cs
        g = plsc.load_gather(tmp_v, [jnp.maximum(lane - k, 0)])
        cs = cs + jnp.where(lane >= k, g, zeros)
    return cs


def _sc_body(tokf_hbm, table_hbm, out_hbm,
             tok_v, pre_v, idx_v, tmp_v, bufs, gsem, ssem):
    ones = jnp.ones((_L,), jnp.int32)
    zeros = jnp.zeros((_L,), jnp.int32)
    lane = lax.iota(jnp.int32, _L)
    sc = lax.axis_index("c")
    sid = lax.axis_index("s")
    wid = sc * _NS + sid
    base = wid * _PER_W
    r = wid % _WPR
    row_start = base - r * _PER_W

    pltpu.sync_copy(tokf_hbm.at[pl.ds(base, _PER_W)], tok_v)
    pltpu.sync_copy(tokf_hbm.at[pl.ds(row_start, _PRE)], pre_v)

    def pre_body(i, acc):
        tv = pre_v[pl.ds(i * _L, _L)]
        return acc + jnp.where(tv != _PAD, ones, zeros)

    tt = lax.fori_loop(0, r * (_PER_W // _L), pre_body, zeros)

    def sweep1(i, acc):
        tv = plsc.load_gather(tok_v, [lane * _SEG + i])
        return acc + jnp.where(tv != _PAD, ones, zeros)

    t = lax.fori_loop(0, _SEG, sweep1, zeros)

    excl = _lane_prefix(t, tmp_v, lane, zeros) - t
    cs_tt = _lane_prefix(tt, tmp_v, lane, zeros)
    tmp_v[...] = cs_tt
    tt_total = plsc.load_gather(tmp_v, [zeros + (_L - 1)])
    start = tt_total + excl

    def sweep2(i, run):
        tv = plsc.load_gather(tok_v, [lane * _SEG + i])
        m = jnp.where(tv != _PAD, ones, zeros)
        run = run + m
        p = lane * _SEG + i
        pos = jnp.minimum(jnp.maximum(run * m + _PAD, 0), _NROW - 1)
        plsc.store_scatter(idx_v, [p], pos)
        return run

    lax.fori_loop(0, _SEG, sweep2, start)

    def g_desc(ci, b):
        off, sz = _CHUNKS[ci]
        return pltpu.make_async_copy(
            table_hbm.at[idx_v.at[pl.ds(off, sz)]],
            bufs.at[b, pl.ds(0, sz)], gsem.at[b])

    def s_desc(ci, b):
        off, sz = _CHUNKS[ci]
        return pltpu.make_async_copy(
            bufs.at[b, pl.ds(0, sz)],
            out_hbm.at[pl.ds(base + off, sz)], ssem.at[b])

    n = len(_CHUNKS)
    g_desc(0, 0).start()
    g_desc(1, 1).start()
    for ci in range(n):
        b = ci % _NBUF
        g_desc(ci, b).wait()
        s_desc(ci, b).start()
        if ci + 2 < n:
            b2 = (ci + 2) % _NBUF
            if ci >= 1:
                s_desc(ci - 1, b2).wait()
            g_desc(ci + 2, b2).start()
    for ci in range(max(0, n - _NBUF), n):
        s_desc(ci, ci % _NBUF).wait()


@functools.cache
def _sc_kernel():
    return pl.kernel(
        _sc_body,
        out_type=jax.ShapeDtypeStruct((_TOT, _D), jnp.float32),
        mesh=plsc.VectorSubcoreMesh(
            core_axis_name="c", subcore_axis_name="s",
            num_cores=_NC, num_subcores=_NS),
        compiler_params=pltpu.CompilerParams(needs_layout_passes=False),
        scratch_types=[
            pltpu.VMEM((_PER_W,), jnp.int32),
            pltpu.VMEM((_PRE,), jnp.int32),
            pltpu.VMEM((_PER_W,), jnp.int32),
            pltpu.VMEM((_L,), jnp.int32),
            pltpu.VMEM((_NBUF, _CH, _D), jnp.float32),
            pltpu.SemaphoreType.DMA((_NBUF,)),
            pltpu.SemaphoreType.DMA((_NBUF,)),
        ],
    )


def kernel(input, weights):
    tokf = input.astype(jnp.int32).reshape(_TOT)
    flat = _sc_kernel()(tokf, weights)
    return flat.reshape(_B, _S, _D)


# speedup vs baseline: 1.0374x; 1.0374x over previous
import functools

import jax
import jax.numpy as jnp
from jax import lax
from jax.experimental import pallas as pl
from jax.experimental.pallas import tpu as pltpu
from jax.experimental.pallas import tpu_sc as plsc

_PAD = 1
_B = 4
_S = 8192
_D = 1024
_TOT = _B * _S
_NROW = 16384

_NC = 2
_NS = 16
_NW = _NC * _NS
_PER_W = _TOT // _NW
_WPR = _S // _PER_W
_L = 16
_SEG = _PER_W // _L
_PRE = (_WPR - 1) * _PER_W
_CH = 32
_NCHUNK = _PER_W // _CH
_NBUF = 3


def _lane_prefix(x, tmp_v, lane, zeros):
    cs = x
    for k in (1, 2, 4, 8):
        tmp_v[...] = cs
        g = plsc.load_gather(tmp_v, [jnp.maximum(lane - k, 0)])
        cs = cs + jnp.where(lane >= k, g, zeros)
    return cs


def _sc_body(tokf_hbm, table_hbm, out_hbm,
             tok_v, pre_v, idx_v, tmp_v, bufs, gsem, ssem):
    ones = jnp.ones((_L,), jnp.int32)
    zeros = jnp.zeros((_L,), jnp.int32)
    lane = lax.iota(jnp.int32, _L)
    sc = lax.axis_index("c")
    sid = lax.axis_index("s")
    wid = sc * _NS + sid
    base = wid * _PER_W
    r = wid % _WPR
    row_start = base - r * _PER_W

    pltpu.sync_copy(tokf_hbm.at[pl.ds(base, _PER_W)], tok_v)
    pltpu.sync_copy(tokf_hbm.at[pl.ds(row_start, _PRE)], pre_v)

    def pre_body(i, acc):
        tv = pre_v[pl.ds(i * _L, _L)]
        return acc + jnp.where(tv != _PAD, ones, zeros)

    tt = lax.fori_loop(0, r * (_PER_W // _L), pre_body, zeros)

    def sweep1(i, acc):
        tv = plsc.load_gather(tok_v, [lane * _SEG + i])
        return acc + jnp.where(tv != _PAD, ones, zeros)

    t = lax.fori_loop(0, _SEG, sweep1, zeros)

    excl = _lane_prefix(t, tmp_v, lane, zeros) - t
    cs_tt = _lane_prefix(tt, tmp_v, lane, zeros)
    tmp_v[...] = cs_tt
    tt_total = plsc.load_gather(tmp_v, [zeros + (_L - 1)])
    start = tt_total + excl

    def sweep2(i, run):
        tv = plsc.load_gather(tok_v, [lane * _SEG + i])
        m = jnp.where(tv != _PAD, ones, zeros)
        run = run + m
        p = lane * _SEG + i
        pos = jnp.minimum(jnp.maximum(run * m + _PAD, 0), _NROW - 1)
        plsc.store_scatter(idx_v, [p >> 5, p & (_CH - 1)], pos)
        return run

    lax.fori_loop(0, _SEG, sweep2, start)

    def g_desc(ci, b):
        return pltpu.make_async_copy(
            table_hbm.at[idx_v.at[ci]], bufs.at[b], gsem.at[b])

    def s_desc(ci, b):
        return pltpu.make_async_copy(
            bufs.at[b], out_hbm.at[pl.ds(base + ci * _CH, _CH)], ssem.at[b])

    g_desc(0, 0).start()
    g_desc(1, 1).start()

    g_desc(0, 0).wait()
    s_desc(0, 0).start()
    g_desc(2, 2).start()
    for ci in (1, 2):
        b = ci % _NBUF
        b2 = (ci + 2) % _NBUF
        g_desc(ci, b).wait()
        s_desc(ci, b).start()
        s_desc(ci - 1, b2).wait()
        g_desc(ci + 2, b2).start()

    def round_body(g, carry):
        for k in range(_NBUF):
            ci = g * _NBUF + k
            b2 = (k + 2) % _NBUF
            g_desc(ci, k).wait()
            s_desc(ci, k).start()
            s_desc(ci - 1, b2).wait()
            g_desc(ci + 2, b2).start()
        return carry

    lax.fori_loop(1, (_NCHUNK - 2) // _NBUF, round_body, 0)

    for ci in (_NCHUNK - 2, _NCHUNK - 1):
        b = ci % _NBUF
        g_desc(ci, b).wait()
        s_desc(ci, b).start()
    for ci in (_NCHUNK - 3, _NCHUNK - 2, _NCHUNK - 1):
        s_desc(ci, ci % _NBUF).wait()


@functools.cache
def _sc_kernel():
    return pl.kernel(
        _sc_body,
        out_type=jax.ShapeDtypeStruct((_TOT, _D), jnp.float32),
        mesh=plsc.VectorSubcoreMesh(
            core_axis_name="c", subcore_axis_name="s",
            num_cores=_NC, num_subcores=_NS),
        compiler_params=pltpu.CompilerParams(needs_layout_passes=False),
        scratch_types=[
            pltpu.VMEM((_PER_W,), jnp.int32),
            pltpu.VMEM((_PRE,), jnp.int32),
            pltpu.VMEM((_NCHUNK, _CH), jnp.int32),
            pltpu.VMEM((_L,), jnp.int32),
            pltpu.VMEM((_NBUF, _CH, _D), jnp.float32),
            pltpu.SemaphoreType.DMA((_NBUF,)),
            pltpu.SemaphoreType.DMA((_NBUF,)),
        ],
    )


def kernel(input, weights):
    tokf = input.astype(jnp.int32).reshape(_TOT)
    flat = _sc_kernel()(tokf, weights)
    return flat.reshape(_B, _S, _D)
